# baseline (speedup 1.0000x reference)
import functools

import jax
import jax.numpy as jnp
from jax import lax
from jax.experimental import pallas as pl
from jax.experimental.pallas import tpu as pltpu
from jax.experimental.pallas import tpu_sc as plsc

N = 248
NP = 256
F = 512
E = 15872
_VEC = 16

_EPC = E // 2
_NVC = _EPC // _VEC
_ROWS = 16
_UNROLL = 4


def _sc_body(edge_hbm, c_hbm, e2_v, blk_v, sem):
    cid = lax.axis_index("c")
    sid = lax.axis_index("s")

    cp = pltpu.make_async_copy(
        edge_hbm.at[:, pl.ds(cid * _EPC, _EPC)], e2_v, sem)
    cp.start()
    zeros = jnp.zeros((_VEC,), jnp.float32)

    def zero_body(r, carry):
        for k in range(NP // _VEC):
            blk_v[r, pl.ds(k * _VEC, _VEC)] = zeros
        return carry

    lax.fori_loop(0, _ROWS, zero_body, 0)
    cp.wait()

    ones = jnp.ones((_VEC,), jnp.float32)

    def edge_body(j, carry):
        for u in range(_UNROLL):
            jj = j * _UNROLL + u
            s = e2_v[0, pl.ds(jj * _VEC, _VEC)]
            d = e2_v[1, pl.ds(jj * _VEC, _VEC)]
            keep = jnp.logical_and((d >> 4) == sid, s != d)
            plsc.addupdate_scatter(blk_v, [d & (_ROWS - 1), s], ones,
                                   mask=keep)
        return carry

    lax.fori_loop(0, _NVC // _UNROLL, edge_body, 0)

    pltpu.sync_copy(blk_v, c_hbm.at[pl.ds(cid * NP + sid * _ROWS, _ROWS), :])


@functools.cache
def _sc_build_c():
    return pl.kernel(
        _sc_body,
        out_type=jax.ShapeDtypeStruct((2 * NP, NP), jnp.float32),
        mesh=plsc.VectorSubcoreMesh(core_axis_name="c", subcore_axis_name="s"),
        compiler_params=pltpu.CompilerParams(needs_layout_passes=False),
        scratch_types=[
            pltpu.VMEM((2, _EPC), jnp.int32),
            pltpu.VMEM((_ROWS, NP), jnp.float32),
            pltpu.SemaphoreType.DMA,
        ],
    )


def _tc_body(c_ref, x_ref, lpw_ref, lpb_ref, w10_ref, w11_ref, b1_ref,
             w20_ref, w21_ref, b2_ref, w30_ref, w31_ref, b3_ref, out_ref):
    f32 = jnp.float32
    C = c_ref[:N, :] + c_ref[NP:NP + N, :]
    deg = jnp.sum(C, axis=0, keepdims=True)
    dis = jnp.where(deg > 0, lax.rsqrt(jnp.maximum(deg, 1e-12)), 0.0)

    rows = lax.broadcasted_iota(jnp.int32, (N, N), 0)
    cols = lax.broadcasted_iota(jnp.int32, (N, N), 1)
    eye = jnp.where(rows == cols, 1.0, 0.0).astype(f32)
    ddiag = eye * dis[:, :N]
    A = -jnp.dot(ddiag, (C * dis)[:, :N], preferred_element_type=f32)

    def mm_t(a, w):
        return lax.dot_general(a, w, (((1,), (1,)), ((), ())),
                               preferred_element_type=f32)

    def silu(h):
        return h / (1.0 + jnp.exp(-h))

    def ln(h):
        mu = jnp.sum(h) / (N * 256)
        dcen = h - mu
        var = jnp.sum(dcen * dcen) / (N * 256)
        return dcen * lax.rsqrt(var + 1e-5)

    x = x_ref[...]
    res = mm_t(x, lpw_ref[...]) + lpb_ref[...]
    t1 = jnp.dot(A, x, preferred_element_type=f32)
    h = mm_t(x, w10_ref[...]) + mm_t(t1, w11_ref[...]) + b1_ref[...]
    h = ln(silu(h))
    t2 = jnp.dot(A, h, preferred_element_type=f32)
    h = res + mm_t(h, w20_ref[...]) + mm_t(t2, w21_ref[...]) + b2_ref[...]
    h = ln(silu(h))
    t3 = jnp.dot(A, h, preferred_element_type=f32)
    out_ref[...] = (mm_t(h, w30_ref[...]) + mm_t(t3, w31_ref[...])
                    + b3_ref[...])


_tc_call = pl.pallas_call(
    _tc_body,
    out_shape=jax.ShapeDtypeStruct((N, 128), jnp.float32),
)


def kernel(x, edge_index, LP_W, LP_b, W1_0, W1_1, b1, W2_0, W2_1, b2,
           W3_0, W3_1, b3):
    c2 = _sc_build_c()(edge_index)
    return _tc_call(c2, x, LP_W, LP_b.reshape(1, -1), W1_0, W1_1,
                    b1.reshape(1, -1), W2_0, W2_1, b2.reshape(1, -1),
                    W3_0, W3_1, b3.reshape(1, -1))


# speedup vs baseline: 1.2137x; 1.2137x over previous
import functools

import jax
import jax.numpy as jnp
from jax import lax
from jax.experimental import pallas as pl
from jax.experimental.pallas import tpu as pltpu
from jax.experimental.pallas import tpu_sc as plsc

N = 248
NP = 256
F = 512
E = 15872
_VEC = 16

_NSUB = 16
_EPT = 512
_NCHUNK = E // _EPT
_NV = _EPT // _VEC
_IDXROWS = 4
_CWP = NP * NP
_ZW = _CWP // _NSUB
_SLAB = _ZW // NP


def _sc_body(edge_hbm, c_hbm, e2_v, idx_v, val_v, zero_v, bflat_v, b2d_v,
             c_sh, sem):
    cid = lax.axis_index("c")
    sid = lax.axis_index("s")
    chunk = cid * _NSUB + sid
    active = chunk < _NCHUNK
    base = jnp.minimum(chunk, _NCHUNK - 1) * _EPT

    @pl.when(active)
    def _():
        pltpu.make_async_copy(
            edge_hbm.at[:, pl.ds(base, _EPT)], e2_v, sem).start()
    zeros = jnp.zeros((_VEC,), jnp.float32)

    def zero_body(i, carry):
        for k in range(8):
            zero_v[pl.ds((i * 8 + k) * _VEC, _VEC)] = zeros
        return carry

    lax.fori_loop(0, _ZW // (8 * _VEC), zero_body, 0)
    pltpu.sync_copy(zero_v, c_sh.at[pl.ds(sid * _ZW, _ZW)])

    @pl.when(active)
    def _():
        pltpu.make_async_copy(
            edge_hbm.at[:, pl.ds(base, _EPT)], e2_v, sem).wait()

        for j in range(_NV):
            s = e2_v[0, pl.ds(j * _VEC, _VEC)]
            d = e2_v[1, pl.ds(j * _VEC, _VEC)]
            row, col = divmod(j * _VEC, 128)
            idx_v[row, pl.ds(col, _VEC)] = (d << 8) | s
            val_v[row, pl.ds(col, _VEC)] = jnp.where(
                s != d, 1.0, 0.0).astype(jnp.float32)

    plsc.subcore_barrier()

    @pl.when(active)
    def _():
        cps = [pltpu.async_copy(val_v.at[j], c_sh.at[idx_v.at[j]], sem,
                                add=True)
               for j in range(_IDXROWS)]
        for cp_j in cps:
            cp_j.wait()

    plsc.subcore_barrier()

    pltpu.sync_copy(c_sh.at[pl.ds(sid * _ZW, _ZW)], bflat_v)

    def row_body(r, carry):
        for cc in range(NP // _VEC):
            b2d_v[r, pl.ds(cc * _VEC, _VEC)] = bflat_v[
                pl.ds(r * NP + cc * _VEC, _VEC)]
        return carry

    lax.fori_loop(0, _SLAB, row_body, 0)
    pltpu.sync_copy(b2d_v, c_hbm.at[pl.ds(cid * NP + sid * _SLAB, _SLAB), :])


@functools.cache
def _sc_build_c():
    return pl.kernel(
        _sc_body,
        out_type=jax.ShapeDtypeStruct((2 * NP, NP), jnp.float32),
        mesh=plsc.VectorSubcoreMesh(core_axis_name="c", subcore_axis_name="s"),
        compiler_params=pltpu.CompilerParams(needs_layout_passes=False),
        scratch_types=[
            pltpu.VMEM((2, _EPT), jnp.int32),
            pltpu.VMEM((_IDXROWS, 128), jnp.int32),
            pltpu.VMEM((_IDXROWS, 128), jnp.float32),
            pltpu.VMEM((_ZW,), jnp.float32),
            pltpu.VMEM((_ZW,), jnp.float32),
            pltpu.VMEM((_SLAB, NP), jnp.float32),
            pltpu.VMEM_SHARED((_CWP,), jnp.float32),
            pltpu.SemaphoreType.DMA,
        ],
    )


def _tc_body(c_ref, x_ref, lpw_ref, lpb_ref, w10_ref, w11_ref, b1_ref,
             w20_ref, w21_ref, b2_ref, w30_ref, w31_ref, b3_ref, out_ref):
    f32 = jnp.float32
    C = c_ref[:N, :] + c_ref[NP:NP + N, :]
    deg = jnp.sum(C, axis=0, keepdims=True)
    dis = jnp.where(deg > 0, lax.rsqrt(jnp.maximum(deg, 1e-12)), 0.0)

    rows = lax.broadcasted_iota(jnp.int32, (N, N), 0)
    cols = lax.broadcasted_iota(jnp.int32, (N, N), 1)
    eye = jnp.where(rows == cols, 1.0, 0.0).astype(f32)
    ddiag = eye * dis[:, :N]
    A = -jnp.dot(ddiag, (C * dis)[:, :N], preferred_element_type=f32)

    def mm_t(a, w):
        return lax.dot_general(a, w, (((1,), (1,)), ((), ())),
                               preferred_element_type=f32)

    def silu(h):
        return h / (1.0 + jnp.exp(-h))

    def ln(h):
        mu = jnp.sum(h) / (N * 256)
        dcen = h - mu
        var = jnp.sum(dcen * dcen) / (N * 256)
        return dcen * lax.rsqrt(var + 1e-5)

    x = x_ref[...]
    res = mm_t(x, lpw_ref[...]) + lpb_ref[...]
    t1 = jnp.dot(A, x, preferred_element_type=f32)
    h = mm_t(x, w10_ref[...]) + mm_t(t1, w11_ref[...]) + b1_ref[...]
    h = ln(silu(h))
    t2 = jnp.dot(A, h, preferred_element_type=f32)
    h = res + mm_t(h, w20_ref[...]) + mm_t(t2, w21_ref[...]) + b2_ref[...]
    h = ln(silu(h))
    t3 = jnp.dot(A, h, preferred_element_type=f32)
    out_ref[...] = (mm_t(h, w30_ref[...]) + mm_t(t3, w31_ref[...])
                    + b3_ref[...])


_tc_call = pl.pallas_call(
    _tc_body,
    out_shape=jax.ShapeDtypeStruct((N, 128), jnp.float32),
)


def kernel(x, edge_index, LP_W, LP_b, W1_0, W1_1, b1, W2_0, W2_1, b2,
           W3_0, W3_1, b3):
    c2 = _sc_build_c()(edge_index)
    return _tc_call(c2, x, LP_W, LP_b.reshape(1, -1), W1_0, W1_1,
                    b1.reshape(1, -1), W2_0, W2_1, b2.reshape(1, -1),
                    W3_0, W3_1, b3.reshape(1, -1))
